# Initial kernel scaffold; baseline (speedup 1.0000x reference)
import jax
import jax.numpy as jnp
from jax.experimental import pallas as pl


def kernel(xyz, params):
    raise NotImplementedError("write your pallas kernel here")


# speedup vs baseline: 2.0282x; 2.0282x over previous
import functools

import jax
import jax.numpy as jnp
import numpy as np
from jax.experimental import pallas as pl

_INV_BN = float(1.0 / np.sqrt(np.float32(1.0) + np.float32(1e-5)))


def _fps_kernel(xyz_ref, out_ref, *, npoint):
    X = xyz_ref[...]
    B, _, N = X.shape
    x0, x1, x2 = X[:, 0, :], X[:, 1, :], X[:, 2, :]
    iota = jax.lax.broadcasted_iota(jnp.int32, (B, N), 1)
    iota_np = jax.lax.broadcasted_iota(jnp.int32, (3, B, npoint), 2)

    def body(i, state):
        dist, far, cent = state
        oh = (iota == far[:, None]).astype(jnp.float32)
        c0 = jnp.sum(x0 * oh, axis=1)
        c1 = jnp.sum(x1 * oh, axis=1)
        c2 = jnp.sum(x2 * oh, axis=1)
        d = (x0 - c0[:, None]) ** 2 + (x1 - c1[:, None]) ** 2 \
            + (x2 - c2[:, None]) ** 2
        dist = jnp.minimum(dist, d)
        far = jnp.argmax(dist, axis=1).astype(jnp.int32)
        c = jnp.stack([c0, c1, c2], axis=0)
        cent = jnp.where(iota_np == i, c[:, :, None], cent)
        return dist, far, cent

    init = (jnp.full((B, N), 1e10, jnp.float32),
            jnp.zeros((B,), jnp.int32),
            jnp.zeros((3, B, npoint), jnp.float32))
    _, _, cent = jax.lax.fori_loop(0, npoint, body, init)
    out_ref[...] = jnp.transpose(cent, (1, 0, 2))


def _fps(xyz_b3n, npoint):
    B = xyz_b3n.shape[0]
    return pl.pallas_call(
        functools.partial(_fps_kernel, npoint=npoint),
        out_shape=jax.ShapeDtypeStruct((B, 3, npoint), jnp.float32),
    )(xyz_b3n)


def _group_mlp_kernel(xyz_ref, newxyz_ref, feats_ref, *refs,
                      nsample, radius, nlayers):
    wrefs = refs[:-1]
    out_ref = refs[-1]
    X = xyz_ref[0]
    C = newxyz_ref[0]
    P = feats_ref[0]
    N = X.shape[1]
    Sc = C.shape[1]

    xn = jnp.sum(X * X, axis=0)
    cn = jnp.sum(C * C, axis=0)
    cross = jax.lax.dot_general(
        C, X, (((0,), (0,)), ((), ())),
        preferred_element_type=jnp.float32)
    sqr = cn[:, None] + xn[None, :] - 2.0 * cross
    mask = sqr <= (radius * radius)
    rank = mask.astype(jnp.float32)
    d = 1
    while d < N:
        shifted = jnp.concatenate(
            [jnp.zeros((Sc, d), jnp.float32), rank[:, :-d]], axis=1)
        rank = rank + shifted
        d *= 2
    cnt = rank[:, -1]

    Fall = jnp.concatenate([X.T, P], axis=1)
    slots = []
    for j in range(nsample):
        oh = jnp.where(mask & (rank == float(j + 1)), 1.0, 0.0)
        slots.append(jax.lax.dot_general(
            oh, Fall, (((1,), (0,)), ((), ())),
            preferred_element_type=jnp.float32,
            precision=jax.lax.Precision.HIGHEST))
    G = jnp.stack(slots, axis=1)

    valid = jax.lax.broadcasted_iota(jnp.int32, (Sc, nsample, 1), 1) \
        < cnt.astype(jnp.int32)[:, None, None]
    G = jnp.where(valid, G, G[:, :1, :])

    gxyz = G[:, :, :3] - jnp.transpose(C)[:, None, :]
    x = jnp.concatenate([gxyz, G[:, :, 3:]], axis=-1)
    x = x.reshape(Sc * nsample, -1)
    for l in range(nlayers):
        W, b, g, e = wrefs[4 * l:4 * l + 4]
        x = jax.lax.dot_general(
            x, W[...], (((1,), (1,)), ((), ())),
            preferred_element_type=jnp.float32)
        x = ((x + b[0]) * _INV_BN) * g[0] + e[0]
        x = jnp.maximum(x, 0.0)
    Co = x.shape[-1]
    out_ref[0] = jnp.max(x.reshape(Sc, nsample, Co), axis=1)


def _group_mlp(xyz_b3n, newxyz_b3s, feats_bnc, params, name,
               nsample, radius, s_chunk):
    B, _, N = xyz_b3n.shape
    S = newxyz_b3s.shape[2]
    Cp = feats_bnc.shape[2]
    nlayers = 0
    wargs, wspecs = [], []
    while (name + '_W' + str(nlayers)) in params:
        j = nlayers
        W = params[name + '_W' + str(j)]
        for arr in (W,
                    params[name + '_b' + str(j)].reshape(1, -1),
                    params[name + '_g' + str(j)].reshape(1, -1),
                    params[name + '_e' + str(j)].reshape(1, -1)):
            wargs.append(arr)
            wspecs.append(pl.BlockSpec(arr.shape, lambda b, s: (0, 0)))
        nlayers += 1
    Cout = wargs[-4].shape[0]

    grid = (B, S // s_chunk)
    return pl.pallas_call(
        functools.partial(_group_mlp_kernel, nsample=nsample,
                          radius=radius, nlayers=nlayers),
        grid=grid,
        in_specs=[
            pl.BlockSpec((1, 3, N), lambda b, s: (b, 0, 0)),
            pl.BlockSpec((1, 3, s_chunk), lambda b, s: (b, 0, s)),
            pl.BlockSpec((1, N, Cp), lambda b, s: (b, 0, 0)),
        ] + wspecs,
        out_specs=pl.BlockSpec((1, s_chunk, Cout), lambda b, s: (b, s, 0)),
        out_shape=jax.ShapeDtypeStruct((B, S, Cout), jnp.float32),
    )(xyz_b3n, newxyz_b3s, feats_bnc, *wargs)


def _head_kernel(xyz_ref, feats_ref, *refs):
    out_ref = refs[-1]
    wrefs = refs[:-1]
    Xc = jnp.transpose(xyz_ref[...], (0, 2, 1))
    P = feats_ref[...]
    B, S, _ = P.shape
    x = jnp.concatenate([Xc, P], axis=-1).reshape(B * S, -1)
    for l in range(3):
        W, b, g, e = wrefs[4 * l:4 * l + 4]
        x = jax.lax.dot_general(
            x, W[...], (((1,), (1,)), ((), ())),
            preferred_element_type=jnp.float32)
        x = ((x + b[0]) * _INV_BN) * g[0] + e[0]
        x = jnp.maximum(x, 0.0)
    x = jnp.max(x.reshape(B, S, -1), axis=1)
    k = 12
    for l in range(2):
        W, b, g, e = wrefs[k:k + 4]
        k += 4
        x = jax.lax.dot_general(
            x, W[...], (((1,), (1,)), ((), ())),
            preferred_element_type=jnp.float32)
        x = ((x + b[0]) * _INV_BN) * g[0] + e[0]
        x = jnp.maximum(x, 0.0)
    W, b = wrefs[k:k + 2]
    x = jax.lax.dot_general(
        x, W[...], (((1,), (1,)), ((), ())),
        preferred_element_type=jnp.float32) + b[0]
    shifted = x - jnp.max(x, axis=-1, keepdims=True)
    out_ref[...] = shifted - jnp.log(
        jnp.sum(jnp.exp(shifted), axis=-1, keepdims=True))


def _head(l2x_b3s, l2p_bsc, params):
    B = l2x_b3s.shape[0]
    wargs = []
    for j in range(3):
        wargs += [params['sa3_W' + str(j)],
                  params['sa3_b' + str(j)].reshape(1, -1),
                  params['sa3_g' + str(j)].reshape(1, -1),
                  params['sa3_e' + str(j)].reshape(1, -1)]
    for nm in ('fc1', 'fc2'):
        wargs += [params[nm + '_W'], params[nm + '_b'].reshape(1, -1),
                  params['bn' + nm[-1] + '_g'].reshape(1, -1),
                  params['bn' + nm[-1] + '_e'].reshape(1, -1)]
    wargs += [params['fc3_W'], params['fc3_b'].reshape(1, -1)]
    return pl.pallas_call(
        _head_kernel,
        out_shape=jax.ShapeDtypeStruct((B, 10), jnp.float32),
    )(l2x_b3s, l2p_bsc, *wargs)


def kernel(xyz, params):
    xyz = jnp.asarray(xyz, jnp.float32)
    B, _, N = xyz.shape

    l1x = _fps(xyz, 512)
    pts0 = jnp.transpose(xyz, (0, 2, 1))
    l1p = _group_mlp(xyz, l1x, pts0, params, 'sa1',
                     nsample=32, radius=0.2, s_chunk=128)

    l2x = _fps(l1x, 128)
    l2p = _group_mlp(l1x, l2x, l1p, params, 'sa2',
                     nsample=64, radius=0.4, s_chunk=128)

    return _head(l2x, l2p, params)


# speedup vs baseline: 2.0522x; 1.0118x over previous
import functools

import jax
import jax.numpy as jnp
import numpy as np
from jax.experimental import pallas as pl

_INV_BN = float(1.0 / np.sqrt(np.float32(1.0) + np.float32(1e-5)))


def _fps_kernel(xyz_ref, out_ref, *, npoint):
    X = xyz_ref[...]
    B, _, N = X.shape
    x0, x1, x2 = X[:, 0, :], X[:, 1, :], X[:, 2, :]
    iota = jax.lax.broadcasted_iota(jnp.int32, (B, N), 1)
    iota_np = jax.lax.broadcasted_iota(jnp.int32, (3, B, npoint), 2)

    def body(i, state):
        dist, far, cent = state
        oh = (iota == far[:, None]).astype(jnp.float32)
        c0 = jnp.sum(x0 * oh, axis=1)
        c1 = jnp.sum(x1 * oh, axis=1)
        c2 = jnp.sum(x2 * oh, axis=1)
        d = (x0 - c0[:, None]) ** 2 + (x1 - c1[:, None]) ** 2 \
            + (x2 - c2[:, None]) ** 2
        dist = jnp.minimum(dist, d)
        far = jnp.argmax(dist, axis=1).astype(jnp.int32)
        c = jnp.stack([c0, c1, c2], axis=0)
        cent = jnp.where(iota_np == i, c[:, :, None], cent)
        return dist, far, cent

    init = (jnp.full((B, N), 1e10, jnp.float32),
            jnp.zeros((B,), jnp.int32),
            jnp.zeros((3, B, npoint), jnp.float32))
    _, _, cent = jax.lax.fori_loop(0, npoint, body, init)
    out_ref[...] = jnp.transpose(cent, (1, 0, 2))


def _fps(xyz_b3n, npoint):
    B = xyz_b3n.shape[0]
    return pl.pallas_call(
        functools.partial(_fps_kernel, npoint=npoint),
        out_shape=jax.ShapeDtypeStruct((B, 3, npoint), jnp.float32),
    )(xyz_b3n)


def _group_mlp_kernel(xyz_ref, newxyz_ref, feats_ref, *refs,
                      nsample, radius, nlayers):
    wrefs = refs[:-1]
    out_ref = refs[-1]
    X = xyz_ref[0]
    C = newxyz_ref[0]
    P = feats_ref[0]
    N = X.shape[1]
    Sc = C.shape[1]

    xn = jnp.sum(X * X, axis=0)
    cn = jnp.sum(C * C, axis=0)
    cross = jax.lax.dot_general(
        C, X, (((0,), (0,)), ((), ())),
        preferred_element_type=jnp.float32)
    sqr = cn[:, None] + xn[None, :] - 2.0 * cross
    mask = sqr <= (radius * radius)
    rank = mask.astype(jnp.float32)
    d = 1
    while d < N:
        shifted = jnp.concatenate(
            [jnp.zeros((Sc, d), jnp.float32), rank[:, :-d]], axis=1)
        rank = rank + shifted
        d *= 2
    cnt = rank[:, -1]
    rankm = jnp.where(mask, rank, 0.0)

    Fall = jnp.concatenate([X.T, P], axis=1)
    slots = []
    for j in range(nsample):
        oh = jnp.where(rankm == float(j + 1), 1.0, 0.0)
        slots.append(jax.lax.dot_general(
            oh, Fall, (((1,), (0,)), ((), ())),
            preferred_element_type=jnp.float32,
            precision=jax.lax.Precision.HIGHEST))
    G = jnp.stack(slots, axis=1)

    valid = jax.lax.broadcasted_iota(jnp.int32, (Sc, nsample, 1), 1) \
        < cnt.astype(jnp.int32)[:, None, None]
    G = jnp.where(valid, G, G[:, :1, :])

    gxyz = G[:, :, :3] - jnp.transpose(C)[:, None, :]
    x = jnp.concatenate([gxyz, G[:, :, 3:]], axis=-1)
    x = x.reshape(Sc * nsample, -1)
    for l in range(nlayers):
        W, b, g, e = wrefs[4 * l:4 * l + 4]
        x = jax.lax.dot_general(
            x, W[...], (((1,), (1,)), ((), ())),
            preferred_element_type=jnp.float32)
        x = ((x + b[0]) * _INV_BN) * g[0] + e[0]
        x = jnp.maximum(x, 0.0)
    Co = x.shape[-1]
    out_ref[0] = jnp.max(x.reshape(Sc, nsample, Co), axis=1)


def _group_mlp(xyz_b3n, newxyz_b3s, feats_bnc, params, name,
               nsample, radius, s_chunk):
    B, _, N = xyz_b3n.shape
    S = newxyz_b3s.shape[2]
    Cp = feats_bnc.shape[2]
    nlayers = 0
    wargs, wspecs = [], []
    while (name + '_W' + str(nlayers)) in params:
        j = nlayers
        W = params[name + '_W' + str(j)]
        for arr in (W,
                    params[name + '_b' + str(j)].reshape(1, -1),
                    params[name + '_g' + str(j)].reshape(1, -1),
                    params[name + '_e' + str(j)].reshape(1, -1)):
            wargs.append(arr)
            wspecs.append(pl.BlockSpec(arr.shape, lambda b, s: (0, 0)))
        nlayers += 1
    Cout = wargs[-4].shape[0]

    grid = (B, S // s_chunk)
    return pl.pallas_call(
        functools.partial(_group_mlp_kernel, nsample=nsample,
                          radius=radius, nlayers=nlayers),
        grid=grid,
        in_specs=[
            pl.BlockSpec((1, 3, N), lambda b, s: (b, 0, 0)),
            pl.BlockSpec((1, 3, s_chunk), lambda b, s: (b, 0, s)),
            pl.BlockSpec((1, N, Cp), lambda b, s: (b, 0, 0)),
        ] + wspecs,
        out_specs=pl.BlockSpec((1, s_chunk, Cout), lambda b, s: (b, s, 0)),
        out_shape=jax.ShapeDtypeStruct((B, S, Cout), jnp.float32),
    )(xyz_b3n, newxyz_b3s, feats_bnc, *wargs)


def _head_kernel(xyz_ref, feats_ref, *refs):
    out_ref = refs[-1]
    wrefs = refs[:-1]
    Xc = jnp.transpose(xyz_ref[...], (0, 2, 1))
    P = feats_ref[...]
    B, S, _ = P.shape
    x = jnp.concatenate([Xc, P], axis=-1).reshape(B * S, -1)
    for l in range(3):
        W, b, g, e = wrefs[4 * l:4 * l + 4]
        x = jax.lax.dot_general(
            x, W[...], (((1,), (1,)), ((), ())),
            preferred_element_type=jnp.float32)
        x = ((x + b[0]) * _INV_BN) * g[0] + e[0]
        x = jnp.maximum(x, 0.0)
    x = jnp.max(x.reshape(B, S, -1), axis=1)
    k = 12
    for l in range(2):
        W, b, g, e = wrefs[k:k + 4]
        k += 4
        x = jax.lax.dot_general(
            x, W[...], (((1,), (1,)), ((), ())),
            preferred_element_type=jnp.float32)
        x = ((x + b[0]) * _INV_BN) * g[0] + e[0]
        x = jnp.maximum(x, 0.0)
    W, b = wrefs[k:k + 2]
    x = jax.lax.dot_general(
        x, W[...], (((1,), (1,)), ((), ())),
        preferred_element_type=jnp.float32) + b[0]
    shifted = x - jnp.max(x, axis=-1, keepdims=True)
    out_ref[...] = shifted - jnp.log(
        jnp.sum(jnp.exp(shifted), axis=-1, keepdims=True))


def _head(l2x_b3s, l2p_bsc, params):
    B = l2x_b3s.shape[0]
    wargs = []
    for j in range(3):
        wargs += [params['sa3_W' + str(j)],
                  params['sa3_b' + str(j)].reshape(1, -1),
                  params['sa3_g' + str(j)].reshape(1, -1),
                  params['sa3_e' + str(j)].reshape(1, -1)]
    for nm in ('fc1', 'fc2'):
        wargs += [params[nm + '_W'], params[nm + '_b'].reshape(1, -1),
                  params['bn' + nm[-1] + '_g'].reshape(1, -1),
                  params['bn' + nm[-1] + '_e'].reshape(1, -1)]
    wargs += [params['fc3_W'], params['fc3_b'].reshape(1, -1)]
    return pl.pallas_call(
        _head_kernel,
        out_shape=jax.ShapeDtypeStruct((B, 10), jnp.float32),
    )(l2x_b3s, l2p_bsc, *wargs)


def kernel(xyz, params):
    xyz = jnp.asarray(xyz, jnp.float32)
    B, _, N = xyz.shape

    l1x = _fps(xyz, 512)
    pts0 = jnp.transpose(xyz, (0, 2, 1))
    l1p = _group_mlp(xyz, l1x, pts0, params, 'sa1',
                     nsample=32, radius=0.2, s_chunk=128)

    l2x = _fps(l1x, 128)
    l2p = _group_mlp(l1x, l2x, l1p, params, 'sa2',
                     nsample=64, radius=0.4, s_chunk=128)

    return _head(l2x, l2p, params)
